# Initial kernel scaffold; baseline (speedup 1.0000x reference)
import jax
import jax.numpy as jnp
from jax.experimental import pallas as pl


def kernel(user_ids, item_ids, user_table, item_table, rows, cols, vals):
    raise NotImplementedError("write your pallas kernel here")


# speedup vs baseline: 3.8414x; 3.8414x over previous
import functools

import jax
import jax.numpy as jnp
from jax import lax
from jax.experimental import pallas as pl
from jax.experimental.pallas import tpu as pltpu
from jax.experimental.pallas import tpu_sc as plsc

NU = 25000
NI = 25000
N = NU + NI
D = 64
E = 800000
EH = E // 2
B = 16384

NC = 2
NS = 16
L = 16

ET = 25088
EHP = ET * NS
EPAD = EHP - EH
CH = 128
NCHUNK = ET // CH

ACC_P = 25088
RPT = ACC_P // NS
PC = 128
PPW = B // (NC * NS)


def _layer_body(emb, rows, cols, vals, out, cbuf, vbuf, rbuf, ribuf, gbuf,
                acc, sem):
  c = lax.axis_index("c")
  s = lax.axis_index("s")

  for t in range(D // L):
    gbuf[0, pl.ds(t * L, L)] = jnp.zeros((L,), jnp.float32)

  def zrow(i, _):
    for j in range(D // L):
      gbuf[i, pl.ds(j * L, L)] = jnp.zeros((L,), jnp.float32)
    return _

  lax.fori_loop(1, CH, zrow, None)
  r0 = s * RPT
  for q in range(RPT // CH):
    pltpu.sync_copy(gbuf, acc.at[pl.ds(r0 + q * CH, CH)])
  rem = RPT - (RPT // CH) * CH
  pltpu.sync_copy(gbuf.at[pl.ds(0, rem)],
                  acc.at[pl.ds(r0 + (RPT // CH) * CH, rem)])
  plsc.subcore_barrier()

  ebase = c * EHP + s * ET
  roff = c * NU

  def chunk(k, _):
    eb = pl.multiple_of(ebase + k * CH, 8)
    pltpu.sync_copy(cols.at[pl.ds(eb, CH)], cbuf)
    pltpu.sync_copy(vals.at[pl.ds(eb, CH)], vbuf)
    pltpu.sync_copy(rows.at[pl.ds(eb, CH)], rbuf)
    pltpu.async_copy(emb.at[cbuf], gbuf, sem).wait()
    for t in range(CH // L):
      ribuf[pl.ds(t * L, L)] = rbuf[pl.ds(t * L, L)] - roff

    def scale16(t, _):
      v16 = vbuf[pl.ds(t * L, L)]
      for el in range(L):
        e = t * L + el
        sv = jnp.zeros((L,), jnp.float32) + v16[el]
        for j in range(D // L):
          gbuf[e, pl.ds(j * L, L)] = gbuf[e, pl.ds(j * L, L)] * sv
      return _

    lax.fori_loop(0, CH // L, scale16, None)
    pltpu.sync_copy(gbuf, acc.at[ribuf], add=True)
    return _

  lax.fori_loop(0, NCHUNK, chunk, None)
  plsc.subcore_barrier()

  g0 = c * NU + s * RPT

  @pl.when(s < NS - 1)
  def _():
    pltpu.sync_copy(acc.at[pl.ds(r0, RPT)], out.at[pl.ds(g0, RPT)])

  @pl.when(s == NS - 1)
  def _():
    last = NU - (NS - 1) * RPT
    pltpu.sync_copy(acc.at[pl.ds(r0, last)], out.at[pl.ds(g0, last)])


def _final_body(l1, l2, l3, uids, iids, out, ubuf, ibuf, gu1, gu2, gu3,
                gi1, gi2, gi3, obuf, sem):
  c = lax.axis_index("c")
  s = lax.axis_index("s")
  w = c * NS + s

  for k in range(PPW // PC):
    b0 = pl.multiple_of(w * PPW + k * PC, 8)
    pltpu.sync_copy(uids.at[pl.ds(b0, PC)], ubuf)
    pltpu.sync_copy(iids.at[pl.ds(b0, PC)], ibuf)
    for t in range(PC // L):
      ibuf[pl.ds(t * L, L)] = ibuf[pl.ds(t * L, L)] + NU
    d1 = pltpu.async_copy(l1.at[ubuf], gu1, sem)
    d2 = pltpu.async_copy(l2.at[ubuf], gu2, sem)
    d3 = pltpu.async_copy(l3.at[ubuf], gu3, sem)
    d4 = pltpu.async_copy(l1.at[ibuf], gi1, sem)
    d5 = pltpu.async_copy(l2.at[ibuf], gi2, sem)
    d6 = pltpu.async_copy(l3.at[ibuf], gi3, sem)
    d1.wait(); d2.wait(); d3.wait(); d4.wait(); d5.wait(); d6.wait()

    lanes = lax.iota(jnp.int32, L)

    def pairs16(t, _):
      res = jnp.zeros((L,), jnp.float32)
      for el in range(L):
        e = t * L + el
        p = jnp.zeros((L,), jnp.float32)
        for j in range(D // L):
          sl = pl.ds(j * L, L)
          su = gu1[e, sl] + gu2[e, sl] + gu3[e, sl]
          si = gi1[e, sl] + gi2[e, sl] + gi3[e, sl]
          p = p + su * si
        res = jnp.where(lanes == el, jnp.sum(p), res)
      obuf[pl.ds(t * L, L)] = res * jnp.float32(1.0 / 9.0)
      return _

    lax.fori_loop(0, PC // L, pairs16, None)
    pltpu.sync_copy(obuf, out.at[pl.ds(b0, PC)])


_mesh = plsc.VectorSubcoreMesh(core_axis_name="c", subcore_axis_name="s",
                               num_cores=NC, num_subcores=NS)

_params = pltpu.CompilerParams(use_tc_tiling_on_sc=False,
                               needs_layout_passes=False)

_layer_fn = functools.partial(
    pl.kernel,
    out_type=jax.ShapeDtypeStruct((N, D), jnp.float32),
    mesh=_mesh,
    scratch_types=[
        pltpu.VMEM((CH,), jnp.int32),
        pltpu.VMEM((CH,), jnp.float32),
        pltpu.VMEM((CH,), jnp.int32),
        pltpu.VMEM((CH,), jnp.int32),
        pltpu.VMEM((CH, D), jnp.float32),
        pltpu.VMEM_SHARED((ACC_P, D), jnp.float32),
        pltpu.SemaphoreType.DMA,
    ],
    compiler_params=_params,
)(_layer_body)

_final_fn = functools.partial(
    pl.kernel,
    out_type=jax.ShapeDtypeStruct((B,), jnp.float32),
    mesh=_mesh,
    scratch_types=[
        pltpu.VMEM((PC,), jnp.int32),
        pltpu.VMEM((PC,), jnp.int32),
        pltpu.VMEM((PC, D), jnp.float32),
        pltpu.VMEM((PC, D), jnp.float32),
        pltpu.VMEM((PC, D), jnp.float32),
        pltpu.VMEM((PC, D), jnp.float32),
        pltpu.VMEM((PC, D), jnp.float32),
        pltpu.VMEM((PC, D), jnp.float32),
        pltpu.VMEM((PC,), jnp.float32),
        pltpu.SemaphoreType.DMA,
    ],
    compiler_params=_params,
)(_final_body)


@jax.jit
def kernel(user_ids, item_ids, user_table, item_table, rows, cols, vals):
  emb0 = jnp.concatenate([user_table, item_table], axis=0)
  zi = jnp.zeros((EPAD,), jnp.int32)
  zf = jnp.zeros((EPAD,), jnp.float32)
  rows_p = jnp.concatenate([rows[:EH], zi, rows[EH:], zi + NU])
  cols_p = jnp.concatenate([cols[:EH], zi, cols[EH:], zi])
  vals_p = jnp.concatenate([vals[:EH], zf, vals[EH:], zf])
  l1 = _layer_fn(emb0, rows_p, cols_p, vals_p)
  l2 = _layer_fn(l1, rows_p, cols_p, vals_p)
  l3 = _layer_fn(l2, rows_p, cols_p, vals_p)
  return _final_fn(l1, l2, l3, user_ids, item_ids)


# speedup vs baseline: 5.9208x; 1.5413x over previous
import functools

import jax
import jax.numpy as jnp
from jax import lax
from jax.experimental import pallas as pl
from jax.experimental.pallas import tpu as pltpu
from jax.experimental.pallas import tpu_sc as plsc

NU = 25000
NI = 25000
N = NU + NI
D = 64
E = 800000
EH = E // 2
B = 16384

NC = 2
NS = 16
L = 16

ET = 25088
EHP = ET * NS
EPAD = EHP - EH
CH = 128
NCHUNK = ET // CH

ACC_P = 25088
RPT = ACC_P // NS
PC = 128
PPW = B // (NC * NS)


def _layer_body(emb, rows2, cols2, vals2, out,
                cbufA, vbufA, ribufA, cbufB, vbufB, ribufB,
                gbufA, gbufB, acc, semEA, semEB, semGA, semGB):
  c = lax.axis_index("c")
  s = lax.axis_index("s")
  w0 = (c * NS + s) * NCHUNK

  def load_edges(k, cb, vb, rb, sem):
    pltpu.async_copy(cols2.at[w0 + k], cb, sem)
    pltpu.async_copy(vals2.at[w0 + k], vb, sem)
    pltpu.async_copy(rows2.at[w0 + k], rb, sem)

  def wait_edges(k, cb, vb, rb, sem):
    pltpu.make_async_copy(cols2.at[w0 + k], cb, sem).wait()
    pltpu.make_async_copy(vals2.at[w0 + k], vb, sem).wait()
    pltpu.make_async_copy(rows2.at[w0 + k], rb, sem).wait()

  def zrow(i, _):
    for j in range(D // L):
      gbufA[i, pl.ds(j * L, L)] = jnp.zeros((L,), jnp.float32)
    return _

  lax.fori_loop(0, CH, zrow, None)
  r0 = s * RPT
  for q in range(RPT // CH):
    pltpu.sync_copy(gbufA, acc.at[pl.ds(r0 + q * CH, CH)])
  rem = RPT - (RPT // CH) * CH
  pltpu.sync_copy(gbufA.at[pl.ds(0, rem)],
                  acc.at[pl.ds(r0 + (RPT // CH) * CH, rem)])
  plsc.subcore_barrier()

  def scale_chunk(vb, gbuf):
    def scale16(t, _):
      v16 = vb[pl.ds(t * L, L)]
      for el in range(L):
        e = t * L + el
        sv = jnp.zeros((L,), jnp.float32) + v16[el]
        for j in range(D // L):
          gbuf[e, pl.ds(j * L, L)] = gbuf[e, pl.ds(j * L, L)] * sv
      return _

    lax.fori_loop(0, CH // L, scale16, None)

  load_edges(0, cbufA, vbufA, ribufA, semEA)
  load_edges(1, cbufB, vbufB, ribufB, semEB)
  wait_edges(0, cbufA, vbufA, ribufA, semEA)
  pltpu.async_copy(emb.at[cbufA], gbufA, semGA)

  def loop(k2, _):
    kA = 2 * k2
    kB = kA + 1
    wait_edges(kB, cbufB, vbufB, ribufB, semEB)
    pltpu.async_copy(emb.at[cbufB], gbufB, semGB)
    pltpu.make_async_copy(emb.at[cbufA], gbufA, semGA).wait()
    scale_chunk(vbufA, gbufA)
    pltpu.sync_copy(gbufA, acc.at[ribufA], add=True)

    @pl.when(kA + 2 < NCHUNK)
    def _():
      load_edges(kA + 2, cbufA, vbufA, ribufA, semEA)

    pltpu.make_async_copy(emb.at[cbufB], gbufB, semGB).wait()
    scale_chunk(vbufB, gbufB)
    pltpu.sync_copy(gbufB, acc.at[ribufB], add=True)

    @pl.when(kB + 2 < NCHUNK)
    def _():
      load_edges(kB + 2, cbufB, vbufB, ribufB, semEB)

    @pl.when(kA + 2 < NCHUNK)
    def _():
      wait_edges(kA + 2, cbufA, vbufA, ribufA, semEA)
      pltpu.async_copy(emb.at[cbufA], gbufA, semGA)

    return _

  lax.fori_loop(0, NCHUNK // 2, loop, None)
  plsc.subcore_barrier()

  g0 = c * NU + s * RPT

  @pl.when(s < NS - 1)
  def _():
    pltpu.sync_copy(acc.at[pl.ds(r0, RPT)], out.at[pl.ds(g0, RPT)])

  @pl.when(s == NS - 1)
  def _():
    last = NU - (NS - 1) * RPT
    pltpu.sync_copy(acc.at[pl.ds(r0, last)], out.at[pl.ds(g0, last)])


def _final_body(l1, l2, l3, uids, iids, out, ubuf, ibuf, gu1, gu2, gu3,
                gi1, gi2, gi3, obuf, sem):
  c = lax.axis_index("c")
  s = lax.axis_index("s")
  w = c * NS + s

  for k in range(PPW // PC):
    b0 = pl.multiple_of(w * PPW + k * PC, 8)
    pltpu.sync_copy(uids.at[pl.ds(b0, PC)], ubuf)
    pltpu.sync_copy(iids.at[pl.ds(b0, PC)], ibuf)
    for t in range(PC // L):
      ibuf[pl.ds(t * L, L)] = ibuf[pl.ds(t * L, L)] + NU
    d1 = pltpu.async_copy(l1.at[ubuf], gu1, sem)
    d2 = pltpu.async_copy(l2.at[ubuf], gu2, sem)
    d3 = pltpu.async_copy(l3.at[ubuf], gu3, sem)
    d4 = pltpu.async_copy(l1.at[ibuf], gi1, sem)
    d5 = pltpu.async_copy(l2.at[ibuf], gi2, sem)
    d6 = pltpu.async_copy(l3.at[ibuf], gi3, sem)
    d1.wait(); d2.wait(); d3.wait(); d4.wait(); d5.wait(); d6.wait()

    lanes = lax.iota(jnp.int32, L)

    def pairs16(t, _):
      res = jnp.zeros((L,), jnp.float32)
      for el in range(L):
        e = t * L + el
        p = jnp.zeros((L,), jnp.float32)
        for j in range(D // L):
          sl = pl.ds(j * L, L)
          su = gu1[e, sl] + gu2[e, sl] + gu3[e, sl]
          si = gi1[e, sl] + gi2[e, sl] + gi3[e, sl]
          p = p + su * si
        res = jnp.where(lanes == el, jnp.sum(p), res)
      obuf[pl.ds(t * L, L)] = res * jnp.float32(1.0 / 9.0)
      return _

    lax.fori_loop(0, PC // L, pairs16, None)
    pltpu.sync_copy(obuf, out.at[pl.ds(b0, PC)])


_mesh = plsc.VectorSubcoreMesh(core_axis_name="c", subcore_axis_name="s",
                               num_cores=NC, num_subcores=NS)

_params = pltpu.CompilerParams(use_tc_tiling_on_sc=False,
                               needs_layout_passes=False)

_layer_fn = functools.partial(
    pl.kernel,
    out_type=jax.ShapeDtypeStruct((N, D), jnp.float32),
    mesh=_mesh,
    scratch_types=[
        pltpu.VMEM((CH,), jnp.int32),
        pltpu.VMEM((CH,), jnp.float32),
        pltpu.VMEM((CH,), jnp.int32),
        pltpu.VMEM((CH,), jnp.int32),
        pltpu.VMEM((CH,), jnp.float32),
        pltpu.VMEM((CH,), jnp.int32),
        pltpu.VMEM((CH, D), jnp.float32),
        pltpu.VMEM((CH, D), jnp.float32),
        pltpu.VMEM_SHARED((ACC_P, D), jnp.float32),
        pltpu.SemaphoreType.DMA,
        pltpu.SemaphoreType.DMA,
        pltpu.SemaphoreType.DMA,
        pltpu.SemaphoreType.DMA,
    ],
    compiler_params=_params,
)(_layer_body)

_final_fn = functools.partial(
    pl.kernel,
    out_type=jax.ShapeDtypeStruct((B,), jnp.float32),
    mesh=_mesh,
    scratch_types=[
        pltpu.VMEM((PC,), jnp.int32),
        pltpu.VMEM((PC,), jnp.int32),
        pltpu.VMEM((PC, D), jnp.float32),
        pltpu.VMEM((PC, D), jnp.float32),
        pltpu.VMEM((PC, D), jnp.float32),
        pltpu.VMEM((PC, D), jnp.float32),
        pltpu.VMEM((PC, D), jnp.float32),
        pltpu.VMEM((PC, D), jnp.float32),
        pltpu.VMEM((PC,), jnp.float32),
        pltpu.SemaphoreType.DMA,
    ],
    compiler_params=_params,
)(_final_body)


@jax.jit
def kernel(user_ids, item_ids, user_table, item_table, rows, cols, vals):
  emb0 = jnp.concatenate([user_table, item_table], axis=0)
  zi = jnp.zeros((EPAD,), jnp.int32)
  zf = jnp.zeros((EPAD,), jnp.float32)
  rows_p = jnp.concatenate([rows[:EH], zi, rows[EH:] - NU, zi])
  cols_p = jnp.concatenate([cols[:EH], zi, cols[EH:], zi])
  vals_p = jnp.concatenate([vals[:EH], zf, vals[EH:], zf])
  rows2 = rows_p.reshape(-1, CH)
  cols2 = cols_p.reshape(-1, CH)
  vals2 = vals_p.reshape(-1, CH)
  l1 = _layer_fn(emb0, rows2, cols2, vals2)
  l2 = _layer_fn(l1, rows2, cols2, vals2)
  l3 = _layer_fn(l2, rows2, cols2, vals2)
  return _final_fn(l1, l2, l3, user_ids, item_ids)
